# baseline (speedup 1.0000x reference)
import functools

import jax
import jax.numpy as jnp
from jax import lax
from jax.experimental import pallas as pl
from jax.experimental.pallas import tpu as pltpu
from jax.experimental.pallas import tpu_sc as plsc

_BATCH = 16384
_DIM = 64
_LANES = 16
_NUM_WORKERS = 32
_BPW = _BATCH // _NUM_WORKERS
_NCH = _BPW // _LANES


def _vec_sqrt(x):
    i = lax.bitcast_convert_type(x, jnp.int32)
    i = jnp.int32(0x1FBD1DF5) + lax.shift_right_logical(i, 1)
    y = lax.bitcast_convert_type(i, jnp.float32)
    for _ in range(3):
        y = 0.5 * (y + x / y)
    return y


def _tec_body(head, relation, tail, ent, rel, out,
              hidx, ridx, tidx, hbuf, rbuf, tbuf, outv, sem):
    wid = lax.axis_index("s") * 2 + lax.axis_index("c")
    base = wid * _BPW

    src = pl.ds(base, _BPW)
    pltpu.sync_copy(head.at[src], hidx)
    pltpu.sync_copy(relation.at[src], ridx)
    pltpu.sync_copy(tail.at[src], tidx)

    row_iota = lax.iota(jnp.int32, _LANES)

    def chunk_body(g, carry):
        sl = pl.ds(g * _LANES, _LANES)
        hv = hidx[sl]
        rv = ridx[sl]
        tv = tidx[sl]
        copies = []
        for j in range(_LANES):
            copies.append(pltpu.async_copy(
                ent.at[pl.ds(hv[j], 1), :], hbuf.at[pl.ds(j, 1), :], sem))
            copies.append(pltpu.async_copy(
                rel.at[pl.ds(rv[j], 1), :], rbuf.at[pl.ds(j, 1), :], sem))
            copies.append(pltpu.async_copy(
                ent.at[pl.ds(tv[j], 1), :], tbuf.at[pl.ds(j, 1), :], sem))
        for cp in copies:
            cp.wait()

        res = jnp.zeros((_LANES,), jnp.float32)
        for e in range(_LANES):
            acc = jnp.zeros((_LANES,), jnp.float32)
            for c in range(_DIM // _LANES):
                csl = pl.ds(c * _LANES, _LANES)
                s = hbuf[e, csl] + rbuf[e, csl] - tbuf[e, csl]
                acc = acc + s * s
            res = jnp.where(row_iota == e, jnp.sum(acc), res)
        outv[sl] = _vec_sqrt(res)
        return carry

    lax.fori_loop(0, _NCH, chunk_body, 0)

    pltpu.sync_copy(outv, out.at[pl.ds(base, _BPW)])


@functools.partial(
    pl.kernel,
    out_type=jax.ShapeDtypeStruct((_BATCH,), jnp.float32),
    mesh=plsc.VectorSubcoreMesh(core_axis_name="c", subcore_axis_name="s"),
    compiler_params=pltpu.CompilerParams(needs_layout_passes=False),
    scratch_types=[
        pltpu.VMEM((_BPW,), jnp.int32),
        pltpu.VMEM((_BPW,), jnp.int32),
        pltpu.VMEM((_BPW,), jnp.int32),
        pltpu.VMEM((_LANES, _DIM), jnp.float32),
        pltpu.VMEM((_LANES, _DIM), jnp.float32),
        pltpu.VMEM((_LANES, _DIM), jnp.float32),
        pltpu.VMEM((_BPW,), jnp.float32),
        pltpu.SemaphoreType.DMA,
    ],
)
def _transe_sc(*args):
    _tec_body(*args)


def kernel(head, relation, tail, entity_table, relation_table):
    res = _transe_sc(head, relation, tail, entity_table, relation_table)
    didx = jnp.repeat(jnp.arange(8192, dtype=jnp.int32) * 64, 2)
    decoy = jnp.take(entity_table, didx, axis=0)
    z = (decoy[0::2, 0] - decoy[1::2, 0]).sum()
    return res + z


# speedup vs baseline: 1.1321x; 1.1321x over previous
import functools

import jax
import jax.numpy as jnp
from jax import lax
from jax.experimental import pallas as pl
from jax.experimental.pallas import tpu as pltpu
from jax.experimental.pallas import tpu_sc as plsc

_BATCH = 16384
_DIM = 64
_LANES = 16
_NUM_WORKERS = 32
_BPW = _BATCH // _NUM_WORKERS
_NCH = _BPW // _LANES


def _vec_sqrt(x):
    i = lax.bitcast_convert_type(x, jnp.int32)
    i = jnp.int32(0x1FBD1DF5) + lax.shift_right_logical(i, 1)
    y = lax.bitcast_convert_type(i, jnp.float32)
    for _ in range(3):
        y = 0.5 * (y + x / y)
    return y


def _tec_body(head, relation, tail, ent, rel, out,
              hidx, ridx, tidx,
              hbufA, rbufA, tbufA, hbufB, rbufB, tbufB, outv, sem):
    wid = lax.axis_index("s") * 2 + lax.axis_index("c")
    base = wid * _BPW

    src = pl.ds(base, _BPW)
    pltpu.sync_copy(head.at[src], hidx)
    pltpu.sync_copy(relation.at[src], ridx)
    pltpu.sync_copy(tail.at[src], tidx)

    row_iota = lax.iota(jnp.int32, _LANES)

    def fire(g, hbuf, rbuf, tbuf):
        sl = pl.ds(g * _LANES, _LANES)
        hv = hidx[sl]
        rv = ridx[sl]
        tv = tidx[sl]
        for j in range(_LANES):
            pltpu.async_copy(
                ent.at[pl.ds(hv[j], 1), :], hbuf.at[pl.ds(j, 1), :], sem)
            pltpu.async_copy(
                rel.at[pl.ds(rv[j], 1), :], rbuf.at[pl.ds(j, 1), :], sem)
            pltpu.async_copy(
                ent.at[pl.ds(tv[j], 1), :], tbuf.at[pl.ds(j, 1), :], sem)

    def drain(hbuf, rbuf, tbuf):
        for j in range(_LANES):
            pltpu.make_async_copy(
                ent.at[pl.ds(0, 1), :], hbuf.at[pl.ds(j, 1), :], sem).wait()
            pltpu.make_async_copy(
                rel.at[pl.ds(0, 1), :], rbuf.at[pl.ds(j, 1), :], sem).wait()
            pltpu.make_async_copy(
                ent.at[pl.ds(0, 1), :], tbuf.at[pl.ds(j, 1), :], sem).wait()

    def compute(g, hbuf, rbuf, tbuf):
        sl = pl.ds(g * _LANES, _LANES)
        res = jnp.zeros((_LANES,), jnp.float32)
        for e in range(_LANES):
            acc = jnp.zeros((_LANES,), jnp.float32)
            for c in range(_DIM // _LANES):
                csl = pl.ds(c * _LANES, _LANES)
                s = hbuf[e, csl] + rbuf[e, csl] - tbuf[e, csl]
                acc = acc + s * s
            res = jnp.where(row_iota == e, jnp.sum(acc), res)
        outv[sl] = _vec_sqrt(res)

    fire(0, hbufA, rbufA, tbufA)

    def body2(k, carry):
        g0 = 2 * k
        fire(g0 + 1, hbufB, rbufB, tbufB)
        drain(hbufA, rbufA, tbufA)
        compute(g0, hbufA, rbufA, tbufA)
        g2 = jnp.where(g0 + 2 < _NCH, g0 + 2, 0)
        fire(g2, hbufA, rbufA, tbufA)
        drain(hbufB, rbufB, tbufB)
        compute(g0 + 1, hbufB, rbufB, tbufB)
        return carry

    lax.fori_loop(0, _NCH // 2, body2, 0)
    drain(hbufA, rbufA, tbufA)

    pltpu.sync_copy(outv, out.at[pl.ds(base, _BPW)])


@functools.partial(
    pl.kernel,
    out_type=jax.ShapeDtypeStruct((_BATCH,), jnp.float32),
    mesh=plsc.VectorSubcoreMesh(core_axis_name="c", subcore_axis_name="s"),
    compiler_params=pltpu.CompilerParams(needs_layout_passes=False),
    scratch_types=[
        pltpu.VMEM((_BPW,), jnp.int32),
        pltpu.VMEM((_BPW,), jnp.int32),
        pltpu.VMEM((_BPW,), jnp.int32),
        pltpu.VMEM((_LANES, _DIM), jnp.float32),
        pltpu.VMEM((_LANES, _DIM), jnp.float32),
        pltpu.VMEM((_LANES, _DIM), jnp.float32),
        pltpu.VMEM((_LANES, _DIM), jnp.float32),
        pltpu.VMEM((_LANES, _DIM), jnp.float32),
        pltpu.VMEM((_LANES, _DIM), jnp.float32),
        pltpu.VMEM((_BPW,), jnp.float32),
        pltpu.SemaphoreType.DMA,
    ],
)
def _transe_sc(*args):
    _tec_body(*args)


def kernel(head, relation, tail, entity_table, relation_table):
    return _transe_sc(head, relation, tail, entity_table, relation_table)


# speedup vs baseline: 1.1337x; 1.0014x over previous
import functools

import jax
import jax.numpy as jnp
from jax import lax
from jax.experimental import pallas as pl
from jax.experimental.pallas import tpu as pltpu
from jax.experimental.pallas import tpu_sc as plsc

_BATCH = 16384
_DIM = 64
_LANES = 16
_NUM_WORKERS = 32
_BPW = _BATCH // _NUM_WORKERS
_NCH = _BPW // _LANES


def _vec_sqrt(x):
    i = lax.bitcast_convert_type(x, jnp.int32)
    i = jnp.int32(0x1FBD1DF5) + lax.shift_right_logical(i, 1)
    y = lax.bitcast_convert_type(i, jnp.float32)
    for _ in range(3):
        y = 0.5 * (y + x / y)
    return y


def _tec_body(head, relation, tail, ent, rel, out,
              hidx, ridx, tidx,
              hbufA, rbufA, tbufA, hbufB, rbufB, tbufB, outv, semA, semB):
    wid = lax.axis_index("s") * 2 + lax.axis_index("c")
    base = wid * _BPW

    src = pl.ds(base, _BPW)
    pltpu.sync_copy(head.at[src], hidx)
    pltpu.sync_copy(relation.at[src], ridx)
    pltpu.sync_copy(tail.at[src], tidx)

    row_iota = lax.iota(jnp.int32, _LANES)

    def fire(g, hbuf, rbuf, tbuf, sem):
        sl = pl.ds(g * _LANES, _LANES)
        hv = hidx[sl]
        rv = ridx[sl]
        tv = tidx[sl]
        for j in range(_LANES):
            pltpu.async_copy(
                ent.at[pl.ds(hv[j], 1), :], hbuf.at[pl.ds(j, 1), :], sem)
            pltpu.async_copy(
                rel.at[pl.ds(rv[j], 1), :], rbuf.at[pl.ds(j, 1), :], sem)
            pltpu.async_copy(
                ent.at[pl.ds(tv[j], 1), :], tbuf.at[pl.ds(j, 1), :], sem)

    def drain(hbuf, rbuf, tbuf, sem):
        for j in range(_LANES):
            pltpu.make_async_copy(
                ent.at[pl.ds(0, 1), :], hbuf.at[pl.ds(j, 1), :], sem).wait()
            pltpu.make_async_copy(
                rel.at[pl.ds(0, 1), :], rbuf.at[pl.ds(j, 1), :], sem).wait()
            pltpu.make_async_copy(
                ent.at[pl.ds(0, 1), :], tbuf.at[pl.ds(j, 1), :], sem).wait()

    def compute(g, hbuf, rbuf, tbuf):
        sl = pl.ds(g * _LANES, _LANES)
        res = jnp.zeros((_LANES,), jnp.float32)
        for e in range(_LANES):
            acc = jnp.zeros((_LANES,), jnp.float32)
            for c in range(_DIM // _LANES):
                csl = pl.ds(c * _LANES, _LANES)
                s = hbuf[e, csl] + rbuf[e, csl] - tbuf[e, csl]
                acc = acc + s * s
            res = jnp.where(row_iota == e, jnp.sum(acc), res)
        outv[sl] = _vec_sqrt(res)

    fire(0, hbufA, rbufA, tbufA, semA)

    def body2(k, carry):
        g0 = 2 * k
        fire(g0 + 1, hbufB, rbufB, tbufB, semB)
        drain(hbufA, rbufA, tbufA, semA)
        compute(g0, hbufA, rbufA, tbufA)
        g2 = jnp.where(g0 + 2 < _NCH, g0 + 2, 0)
        fire(g2, hbufA, rbufA, tbufA, semA)
        drain(hbufB, rbufB, tbufB, semB)
        compute(g0 + 1, hbufB, rbufB, tbufB)
        return carry

    lax.fori_loop(0, _NCH // 2, body2, 0)
    drain(hbufA, rbufA, tbufA, semA)

    pltpu.sync_copy(outv, out.at[pl.ds(base, _BPW)])


@functools.partial(
    pl.kernel,
    out_type=jax.ShapeDtypeStruct((_BATCH,), jnp.float32),
    mesh=plsc.VectorSubcoreMesh(core_axis_name="c", subcore_axis_name="s"),
    compiler_params=pltpu.CompilerParams(needs_layout_passes=False),
    scratch_types=[
        pltpu.VMEM((_BPW,), jnp.int32),
        pltpu.VMEM((_BPW,), jnp.int32),
        pltpu.VMEM((_BPW,), jnp.int32),
        pltpu.VMEM((_LANES, _DIM), jnp.float32),
        pltpu.VMEM((_LANES, _DIM), jnp.float32),
        pltpu.VMEM((_LANES, _DIM), jnp.float32),
        pltpu.VMEM((_LANES, _DIM), jnp.float32),
        pltpu.VMEM((_LANES, _DIM), jnp.float32),
        pltpu.VMEM((_LANES, _DIM), jnp.float32),
        pltpu.VMEM((_BPW,), jnp.float32),
        pltpu.SemaphoreType.DMA,
        pltpu.SemaphoreType.DMA,
    ],
)
def _transe_sc(*args):
    _tec_body(*args)


def kernel(head, relation, tail, entity_table, relation_table):
    return _transe_sc(head, relation, tail, entity_table, relation_table)


# speedup vs baseline: 1.1388x; 1.0045x over previous
import functools

import jax
import jax.numpy as jnp
from jax import lax
from jax.experimental import pallas as pl
from jax.experimental.pallas import tpu as pltpu
from jax.experimental.pallas import tpu_sc as plsc

_BATCH = 16384
_DIM = 64
_LANES = 16
_NUM_WORKERS = 32
_BPW = _BATCH // _NUM_WORKERS
_NCH = _BPW // _LANES


def _vec_sqrt(x):
    i = lax.bitcast_convert_type(x, jnp.int32)
    i = jnp.int32(0x1FBD1DF5) + lax.shift_right_logical(i, 1)
    y = lax.bitcast_convert_type(i, jnp.float32)
    for _ in range(3):
        y = 0.5 * (y + x / y)
    return y


def _tec_body(head, relation, tail, ent, rel, out,
              hidx, ridx, tidx,
              hbufA, rbufA, tbufA, hbufB, rbufB, tbufB, outv, semA, semB):
    wid = lax.axis_index("s") * 2 + lax.axis_index("c")
    base = wid * _BPW

    src = pl.ds(base, _BPW)
    pltpu.sync_copy(head.at[src], hidx)
    pltpu.sync_copy(relation.at[src], ridx)
    pltpu.sync_copy(tail.at[src], tidx)

    row_iota = lax.iota(jnp.int32, _LANES)

    def fire(g, hbuf, rbuf, tbuf, sem):
        sl = pl.ds(g * _LANES, _LANES)
        hv = hidx[sl]
        rv = ridx[sl]
        tv = tidx[sl]
        for j in range(_LANES):
            pltpu.async_copy(
                ent.at[pl.ds(hv[j], 1), :], hbuf.at[pl.ds(j, 1), :], sem)
            pltpu.async_copy(
                rel.at[pl.ds(rv[j], 1), :], rbuf.at[pl.ds(j, 1), :], sem)
            pltpu.async_copy(
                ent.at[pl.ds(tv[j], 1), :], tbuf.at[pl.ds(j, 1), :], sem)

    def drain(hbuf, rbuf, tbuf, sem):
        pltpu.make_async_copy(ent.at[pl.ds(0, _LANES), :], hbuf, sem).wait()
        pltpu.make_async_copy(ent.at[pl.ds(0, _LANES), :], rbuf, sem).wait()
        pltpu.make_async_copy(ent.at[pl.ds(0, _LANES), :], tbuf, sem).wait()

    def compute(g, hbuf, rbuf, tbuf):
        sl = pl.ds(g * _LANES, _LANES)
        res = jnp.zeros((_LANES,), jnp.float32)
        for e in range(_LANES):
            acc = jnp.zeros((_LANES,), jnp.float32)
            for c in range(_DIM // _LANES):
                csl = pl.ds(c * _LANES, _LANES)
                s = hbuf[e, csl] + rbuf[e, csl] - tbuf[e, csl]
                acc = acc + s * s
            res = jnp.where(row_iota == e, jnp.sum(acc), res)
        outv[sl] = _vec_sqrt(res)

    fire(0, hbufA, rbufA, tbufA, semA)

    def body2(k, carry):
        g0 = 2 * k
        fire(g0 + 1, hbufB, rbufB, tbufB, semB)
        drain(hbufA, rbufA, tbufA, semA)
        compute(g0, hbufA, rbufA, tbufA)
        g2 = jnp.where(g0 + 2 < _NCH, g0 + 2, 0)
        fire(g2, hbufA, rbufA, tbufA, semA)
        drain(hbufB, rbufB, tbufB, semB)
        compute(g0 + 1, hbufB, rbufB, tbufB)
        return carry

    lax.fori_loop(0, _NCH // 2, body2, 0)
    drain(hbufA, rbufA, tbufA, semA)

    pltpu.sync_copy(outv, out.at[pl.ds(base, _BPW)])


@functools.partial(
    pl.kernel,
    out_type=jax.ShapeDtypeStruct((_BATCH,), jnp.float32),
    mesh=plsc.VectorSubcoreMesh(core_axis_name="c", subcore_axis_name="s"),
    compiler_params=pltpu.CompilerParams(needs_layout_passes=False),
    scratch_types=[
        pltpu.VMEM((_BPW,), jnp.int32),
        pltpu.VMEM((_BPW,), jnp.int32),
        pltpu.VMEM((_BPW,), jnp.int32),
        pltpu.VMEM((_LANES, _DIM), jnp.float32),
        pltpu.VMEM((_LANES, _DIM), jnp.float32),
        pltpu.VMEM((_LANES, _DIM), jnp.float32),
        pltpu.VMEM((_LANES, _DIM), jnp.float32),
        pltpu.VMEM((_LANES, _DIM), jnp.float32),
        pltpu.VMEM((_LANES, _DIM), jnp.float32),
        pltpu.VMEM((_BPW,), jnp.float32),
        pltpu.SemaphoreType.DMA,
        pltpu.SemaphoreType.DMA,
    ],
)
def _transe_sc(*args):
    _tec_body(*args)


def kernel(head, relation, tail, entity_table, relation_table):
    return _transe_sc(head, relation, tail, entity_table, relation_table)
